# baseline (speedup 1.0000x reference)
import math

import jax
import jax.numpy as jnp
from jax.experimental import pallas as pl
from jax.experimental.pallas import tpu as pltpu

_SCALE = 400.0 / math.log(10.0)
_DEFAULT_PRED = 7.6699353278706015
_NEG = -1e30

_TOTAL = 32768
_D = 256
_H = 32
_B = 16
_NSTRIPE = 4
_STRIPE = _TOTAL // _NSTRIPE
_BLK = 512
_GRID = _STRIPE // _BLK


def _fused_kernel(x0_ref, x1_ref, x2_ref, x3_ref, w1_ref, wrzt_ref, meta_ref,
                  out_ref, acc_ref):
    i = pl.program_id(0)

    @pl.when(i == 0)
    def _init():
        acc_ref[:, 0:1] = jnp.full((_B, 1), _NEG, jnp.float32)
        acc_ref[:, 1:2] = jnp.zeros((_B, 1), jnp.float32)
        acc_ref[:, 2:3] = jnp.zeros((_B, 1), jnp.float32)

    b1 = meta_ref[0:_H, 0:1]
    brz = meta_ref[0:2, 1:2]
    starts = meta_ref[0:_B, 2:3]
    ends = meta_ref[0:_B, 3:4]

    zs = []
    rs = []
    masks = []
    for k, x_ref in enumerate((x0_ref, x1_ref, x2_ref, x3_ref)):
        ht = jnp.maximum(
            jax.lax.dot_general(
                w1_ref[:], x_ref[:], (((1,), (1,)), ((), ())),
                preferred_element_type=jnp.float32,
            ) + b1,
            0.0,
        )
        rzt = jax.lax.dot_general(
            wrzt_ref[:], ht, (((1,), (0,)), ((), ())),
            preferred_element_type=jnp.float32,
        ) + brz
        rs.append(rzt[0:1, :])
        zs.append(rzt[1:2, :])
        idx = (
            jax.lax.broadcasted_iota(jnp.int32, (1, _BLK), 1)
            + (i * _BLK + k * _STRIPE)
        ).astype(jnp.float32)
        masks.append((idx >= starts) & (idx < ends))

    r = jnp.concatenate(rs, axis=1)
    z = jnp.concatenate(zs, axis=1)
    mask = jnp.concatenate(masks, axis=1)
    zm = jnp.where(mask, z, _NEG)

    old_max = acc_ref[:, 0:1]
    blk_max = jnp.max(zm, axis=1, keepdims=True)
    new_max = jnp.maximum(old_max, blk_max)
    scale = jnp.exp(old_max - new_max)

    e = jnp.exp(zm - new_max) * mask.astype(jnp.float32)
    s = jnp.sum(e, axis=1, keepdims=True)
    sr = jnp.sum(e * r, axis=1, keepdims=True)

    acc_ref[:, 0:1] = new_max
    acc_ref[:, 1:2] = acc_ref[:, 1:2] * scale + s
    acc_ref[:, 2:3] = acc_ref[:, 2:3] * scale + sr

    @pl.when(i == _GRID - 1)
    def _finish():
        denom = acc_ref[:, 1:2]
        preds = acc_ref[:, 2:3] / jnp.where(denom == 0.0, 1.0, denom)
        empty = starts == ends
        preds = jnp.where(empty, _DEFAULT_PRED, preds)
        out_ref[:] = _SCALE * preds


def kernel(x, xlens, W1, b1, Wr, br, Wz, bz):
    wrzt = jnp.concatenate([Wr, Wz], axis=0)

    clens = jnp.concatenate(
        [jnp.zeros((1,), dtype=xlens.dtype), jnp.cumsum(xlens)]
    )
    starts = clens[:-1].astype(jnp.float32)
    ends = clens[1:].astype(jnp.float32)

    meta = jnp.zeros((_H, 8), jnp.float32)
    meta = meta.at[:, 0].set(b1)
    meta = meta.at[0, 1].set(br[0])
    meta = meta.at[1, 1].set(bz[0])
    meta = meta.at[:_B, 2].set(starts)
    meta = meta.at[:_B, 3].set(ends)

    stripes = [x[k * _STRIPE:(k + 1) * _STRIPE] for k in range(_NSTRIPE)]

    out = pl.pallas_call(
        _fused_kernel,
        grid=(_GRID,),
        in_specs=[pl.BlockSpec((_BLK, _D), lambda i: (i, 0))] * _NSTRIPE + [
            pl.BlockSpec((_H, _D), lambda i: (0, 0)),
            pl.BlockSpec((2, _H), lambda i: (0, 0)),
            pl.BlockSpec((_H, 8), lambda i: (0, 0)),
        ],
        out_specs=pl.BlockSpec((_B, 1), lambda i: (0, 0)),
        out_shape=jax.ShapeDtypeStruct((_B, 1), jnp.float32),
        scratch_shapes=[pltpu.VMEM((_B, 8), jnp.float32)],
    )(*stripes, W1, wrzt, meta)
    return out.reshape(_B)


# speedup vs baseline: 2.0327x; 2.0327x over previous
import math

import jax
import jax.numpy as jnp
from jax.experimental import pallas as pl
from jax.experimental.pallas import tpu as pltpu

_SCALE = 400.0 / math.log(10.0)
_DEFAULT_PRED = 7.6699353278706015
_NEG = -1e30

_TOTAL = 32768
_D = 256
_H = 32
_B = 16
_BLK = 2048
_GRID = _TOTAL // _BLK


def _fused_kernel(x_ref, w1t_ref, wrz_ref, meta_ref, out_ref, acc_ref):
    i = pl.program_id(0)

    @pl.when(i == 0)
    def _init():
        acc_ref[:, 0:1] = jnp.full((_B, 1), _NEG, jnp.float32)
        acc_ref[:, 1:2] = jnp.zeros((_B, 1), jnp.float32)
        acc_ref[:, 2:3] = jnp.zeros((_B, 1), jnp.float32)

    xb = x_ref[:]
    b1 = meta_ref[0:1, 0:_H]
    brz = meta_ref[1:2, 0:2]
    starts = meta_ref[2:3, 0:_B].reshape(_B, 1)
    ends = meta_ref[3:4, 0:_B].reshape(_B, 1)

    hb = jnp.maximum(
        jnp.dot(xb, w1t_ref[:], preferred_element_type=jnp.float32) + b1, 0.0
    )
    rz = jnp.dot(hb, wrz_ref[:], preferred_element_type=jnp.float32) + brz
    rzt = rz.T
    r = rzt[0:1, :]
    z = rzt[1:2, :]

    idx = (
        jax.lax.broadcasted_iota(jnp.int32, (1, _BLK), 1) + i * _BLK
    ).astype(jnp.float32)
    mask = (idx >= starts) & (idx < ends)
    zm = jnp.where(mask, z, _NEG)

    old_max = acc_ref[:, 0:1]
    blk_max = jnp.max(zm, axis=1, keepdims=True)
    new_max = jnp.maximum(old_max, blk_max)
    scale = jnp.exp(old_max - new_max)

    e = jnp.exp(zm - new_max) * mask.astype(jnp.float32)
    s = jnp.sum(e, axis=1, keepdims=True)
    sr = jnp.sum(e * r, axis=1, keepdims=True)

    acc_ref[:, 0:1] = new_max
    acc_ref[:, 1:2] = acc_ref[:, 1:2] * scale + s
    acc_ref[:, 2:3] = acc_ref[:, 2:3] * scale + sr

    @pl.when(i == _GRID - 1)
    def _finish():
        denom = acc_ref[:, 1:2]
        preds = acc_ref[:, 2:3] / jnp.where(denom == 0.0, 1.0, denom)
        empty = starts == ends
        preds = jnp.where(empty, _DEFAULT_PRED, preds)
        out_ref[:] = _SCALE * preds


def kernel(x, xlens, W1, b1, Wr, br, Wz, bz):
    w1t = W1.T
    wrz = jnp.concatenate([Wr, Wz], axis=0).T

    clens = jnp.concatenate(
        [jnp.zeros((1,), dtype=xlens.dtype), jnp.cumsum(xlens)]
    )
    starts = clens[:-1].astype(jnp.float32)
    ends = clens[1:].astype(jnp.float32)

    meta = jnp.zeros((8, _H), jnp.float32)
    meta = meta.at[0, :].set(b1)
    meta = meta.at[1, 0].set(br[0])
    meta = meta.at[1, 1].set(bz[0])
    meta = meta.at[2, :_B].set(starts)
    meta = meta.at[3, :_B].set(ends)

    out = pl.pallas_call(
        _fused_kernel,
        grid=(_GRID,),
        in_specs=[
            pl.BlockSpec((_BLK, _D), lambda i: (i, 0)),
            pl.BlockSpec((_D, _H), lambda i: (0, 0)),
            pl.BlockSpec((_H, 2), lambda i: (0, 0)),
            pl.BlockSpec((8, _H), lambda i: (0, 0)),
        ],
        out_specs=pl.BlockSpec((_B, 1), lambda i: (0, 0)),
        out_shape=jax.ShapeDtypeStruct((_B, 1), jnp.float32),
        scratch_shapes=[pltpu.VMEM((_B, 8), jnp.float32)],
    )(x, w1t, wrz, meta)
    return out.reshape(_B)


# speedup vs baseline: 2.4844x; 1.2222x over previous
import math

import jax
import jax.numpy as jnp
from jax.experimental import pallas as pl
from jax.experimental.pallas import tpu as pltpu

_SCALE = 400.0 / math.log(10.0)
_DEFAULT_PRED = 7.6699353278706015
_NEG = -1e30

_TOTAL = 32768
_D = 256
_H = 32
_B = 16
_BLK = 4096
_GRID = _TOTAL // _BLK


def _fused_kernel(x_ref, w1t_ref, wrz_ref, meta_ref, out_ref, acc_ref):
    i = pl.program_id(0)

    @pl.when(i == 0)
    def _init():
        acc_ref[:, 0:1] = jnp.full((_B, 1), _NEG, jnp.float32)
        acc_ref[:, 1:2] = jnp.zeros((_B, 1), jnp.float32)
        acc_ref[:, 2:3] = jnp.zeros((_B, 1), jnp.float32)

    xb = x_ref[:]
    b1 = meta_ref[0:1, 0:_H]
    brz = meta_ref[1:2, 0:2]
    starts = meta_ref[2:3, 0:_B].reshape(_B, 1)
    ends = meta_ref[3:4, 0:_B].reshape(_B, 1)

    hb = jnp.maximum(
        jnp.dot(xb, w1t_ref[:], preferred_element_type=jnp.float32) + b1, 0.0
    )
    rz = jnp.dot(hb, wrz_ref[:], preferred_element_type=jnp.float32) + brz
    rzt = rz.T
    r = rzt[0:1, :]
    z = rzt[1:2, :]

    idx = (
        jax.lax.broadcasted_iota(jnp.int32, (1, _BLK), 1) + i * _BLK
    ).astype(jnp.float32)
    mask = (idx >= starts) & (idx < ends)
    zm = jnp.where(mask, z, _NEG)

    old_max = acc_ref[:, 0:1]
    blk_max = jnp.max(zm, axis=1, keepdims=True)
    new_max = jnp.maximum(old_max, blk_max)
    scale = jnp.exp(old_max - new_max)

    e = jnp.exp(zm - new_max) * mask.astype(jnp.float32)
    s = jnp.sum(e, axis=1, keepdims=True)
    sr = jnp.sum(e * r, axis=1, keepdims=True)

    acc_ref[:, 0:1] = new_max
    acc_ref[:, 1:2] = acc_ref[:, 1:2] * scale + s
    acc_ref[:, 2:3] = acc_ref[:, 2:3] * scale + sr

    @pl.when(i == _GRID - 1)
    def _finish():
        denom = acc_ref[:, 1:2]
        preds = acc_ref[:, 2:3] / jnp.where(denom == 0.0, 1.0, denom)
        empty = starts == ends
        preds = jnp.where(empty, _DEFAULT_PRED, preds)
        out_ref[:] = _SCALE * preds


def kernel(x, xlens, W1, b1, Wr, br, Wz, bz):
    w1t = W1.T
    wrz = jnp.concatenate([Wr, Wz], axis=0).T

    clens = jnp.concatenate(
        [jnp.zeros((1,), dtype=xlens.dtype), jnp.cumsum(xlens)]
    )
    starts = clens[:-1].astype(jnp.float32)
    ends = clens[1:].astype(jnp.float32)

    meta = jnp.zeros((8, _H), jnp.float32)
    meta = meta.at[0, :].set(b1)
    meta = meta.at[1, 0].set(br[0])
    meta = meta.at[1, 1].set(bz[0])
    meta = meta.at[2, :_B].set(starts)
    meta = meta.at[3, :_B].set(ends)

    out = pl.pallas_call(
        _fused_kernel,
        grid=(_GRID,),
        in_specs=[
            pl.BlockSpec((_BLK, _D), lambda i: (i, 0)),
            pl.BlockSpec((_D, _H), lambda i: (0, 0)),
            pl.BlockSpec((_H, 2), lambda i: (0, 0)),
            pl.BlockSpec((8, _H), lambda i: (0, 0)),
        ],
        out_specs=pl.BlockSpec((_B, 1), lambda i: (0, 0)),
        out_shape=jax.ShapeDtypeStruct((_B, 1), jnp.float32),
        scratch_shapes=[pltpu.VMEM((_B, 8), jnp.float32)],
    )(x, w1t, wrz, meta)
    return out.reshape(_B)


# speedup vs baseline: 2.6711x; 1.0751x over previous
import math

import jax
import jax.numpy as jnp
from jax.experimental import pallas as pl
from jax.experimental.pallas import tpu as pltpu

_SCALE = 400.0 / math.log(10.0)
_DEFAULT_PRED = 7.6699353278706015
_NEG = -1e30

_TOTAL = 32768
_D = 256
_H = 32
_B = 16
_BLK = 8192
_GRID = _TOTAL // _BLK


def _fused_kernel(x_ref, w1t_ref, wrz_ref, meta_ref, out_ref, acc_ref):
    i = pl.program_id(0)

    @pl.when(i == 0)
    def _init():
        acc_ref[:, 0:1] = jnp.full((_B, 1), _NEG, jnp.float32)
        acc_ref[:, 1:2] = jnp.zeros((_B, 1), jnp.float32)
        acc_ref[:, 2:3] = jnp.zeros((_B, 1), jnp.float32)

    xb = x_ref[:]
    b1 = meta_ref[0:1, 0:_H]
    brz = meta_ref[1:2, 0:2]
    starts = meta_ref[2:3, 0:_B].reshape(_B, 1)
    ends = meta_ref[3:4, 0:_B].reshape(_B, 1)

    hb = jnp.maximum(
        jnp.dot(xb, w1t_ref[:], preferred_element_type=jnp.float32) + b1, 0.0
    )
    rz = jnp.dot(hb, wrz_ref[:], preferred_element_type=jnp.float32) + brz
    rzt = rz.T
    r = rzt[0:1, :]
    z = rzt[1:2, :]

    idx = (
        jax.lax.broadcasted_iota(jnp.int32, (1, _BLK), 1) + i * _BLK
    ).astype(jnp.float32)
    mask = (idx >= starts) & (idx < ends)
    zm = jnp.where(mask, z, _NEG)

    old_max = acc_ref[:, 0:1]
    blk_max = jnp.max(zm, axis=1, keepdims=True)
    new_max = jnp.maximum(old_max, blk_max)
    scale = jnp.exp(old_max - new_max)

    e = jnp.exp(zm - new_max) * mask.astype(jnp.float32)
    s = jnp.sum(e, axis=1, keepdims=True)
    sr = jnp.sum(e * r, axis=1, keepdims=True)

    acc_ref[:, 0:1] = new_max
    acc_ref[:, 1:2] = acc_ref[:, 1:2] * scale + s
    acc_ref[:, 2:3] = acc_ref[:, 2:3] * scale + sr

    @pl.when(i == _GRID - 1)
    def _finish():
        denom = acc_ref[:, 1:2]
        preds = acc_ref[:, 2:3] / jnp.where(denom == 0.0, 1.0, denom)
        empty = starts == ends
        preds = jnp.where(empty, _DEFAULT_PRED, preds)
        out_ref[:] = _SCALE * preds


def kernel(x, xlens, W1, b1, Wr, br, Wz, bz):
    w1t = W1.T
    wrz = jnp.concatenate([Wr, Wz], axis=0).T

    clens = jnp.concatenate(
        [jnp.zeros((1,), dtype=xlens.dtype), jnp.cumsum(xlens)]
    )
    starts = clens[:-1].astype(jnp.float32)
    ends = clens[1:].astype(jnp.float32)

    meta = jnp.zeros((8, _H), jnp.float32)
    meta = meta.at[0, :].set(b1)
    meta = meta.at[1, 0].set(br[0])
    meta = meta.at[1, 1].set(bz[0])
    meta = meta.at[2, :_B].set(starts)
    meta = meta.at[3, :_B].set(ends)

    out = pl.pallas_call(
        _fused_kernel,
        grid=(_GRID,),
        in_specs=[
            pl.BlockSpec((_BLK, _D), lambda i: (i, 0)),
            pl.BlockSpec((_D, _H), lambda i: (0, 0)),
            pl.BlockSpec((_H, 2), lambda i: (0, 0)),
            pl.BlockSpec((8, _H), lambda i: (0, 0)),
        ],
        out_specs=pl.BlockSpec((_B, 1), lambda i: (0, 0)),
        out_shape=jax.ShapeDtypeStruct((_B, 1), jnp.float32),
        scratch_shapes=[pltpu.VMEM((_B, 8), jnp.float32)],
    )(x, w1t, wrz, meta)
    return out.reshape(_B)
